# Initial kernel scaffold; baseline (speedup 1.0000x reference)
import jax
import jax.numpy as jnp
from jax.experimental import pallas as pl


def kernel(x):
    raise NotImplementedError("write your pallas kernel here")


# speedup vs baseline: 11.1154x; 11.1154x over previous
import jax
import jax.numpy as jnp
from jax.experimental import pallas as pl

_ROWS = 320000
_COLS = 128
_BLOCK_ROWS = 4000


def _sum_block_kernel(x_ref, o_ref):
    i = pl.program_id(0)

    @pl.when(i == 0)
    def _init():
        o_ref[...] = jnp.zeros_like(o_ref)

    o_ref[...] += jnp.sum(x_ref[...], axis=0, keepdims=True)


def kernel(x):
    grid = _ROWS // _BLOCK_ROWS
    out = pl.pallas_call(
        _sum_block_kernel,
        grid=(grid,),
        in_specs=[pl.BlockSpec((_BLOCK_ROWS, _COLS), lambda i: (i, 0))],
        out_specs=pl.BlockSpec((1, _COLS), lambda i: (0, 0)),
        out_shape=jax.ShapeDtypeStruct((1, _COLS), jnp.float32),
    )(x)
    return out


# speedup vs baseline: 14.4770x; 1.3024x over previous
import jax
import jax.numpy as jnp
from jax.experimental import pallas as pl

_ROWS = 320000
_COLS = 128
_BLOCK_ROWS = 8000


def _sum_block_kernel(x_ref, o_ref):
    i = pl.program_id(0)

    @pl.when(i == 0)
    def _init():
        o_ref[...] = jnp.zeros_like(o_ref)

    o_ref[...] += jnp.sum(x_ref[...], axis=0, keepdims=True)


def kernel(x):
    grid = _ROWS // _BLOCK_ROWS
    out = pl.pallas_call(
        _sum_block_kernel,
        grid=(grid,),
        in_specs=[pl.BlockSpec((_BLOCK_ROWS, _COLS), lambda i: (i, 0))],
        out_specs=pl.BlockSpec((1, _COLS), lambda i: (0, 0)),
        out_shape=jax.ShapeDtypeStruct((1, _COLS), jnp.float32),
    )(x)
    return out


# speedup vs baseline: 16.8955x; 1.1671x over previous
import jax
import jax.numpy as jnp
from jax.experimental import pallas as pl

_ROWS = 320000
_COLS = 128
_BLOCK_ROWS = 16000


def _sum_block_kernel(x_ref, o_ref):
    i = pl.program_id(0)

    @pl.when(i == 0)
    def _init():
        o_ref[...] = jnp.zeros_like(o_ref)

    o_ref[...] += jnp.sum(x_ref[...], axis=0, keepdims=True)


def kernel(x):
    grid = _ROWS // _BLOCK_ROWS
    out = pl.pallas_call(
        _sum_block_kernel,
        grid=(grid,),
        in_specs=[pl.BlockSpec((_BLOCK_ROWS, _COLS), lambda i: (i, 0))],
        out_specs=pl.BlockSpec((1, _COLS), lambda i: (0, 0)),
        out_shape=jax.ShapeDtypeStruct((1, _COLS), jnp.float32),
    )(x)
    return out


# speedup vs baseline: 17.9352x; 1.0615x over previous
import jax
import jax.numpy as jnp
from jax.experimental import pallas as pl

_ROWS = 320000
_COLS = 128
_BLOCK_ROWS = 32000


def _sum_block_kernel(x_ref, o_ref):
    i = pl.program_id(0)

    @pl.when(i == 0)
    def _init():
        o_ref[...] = jnp.zeros_like(o_ref)

    o_ref[...] += jnp.sum(x_ref[...], axis=0, keepdims=True)


def kernel(x):
    grid = _ROWS // _BLOCK_ROWS
    out = pl.pallas_call(
        _sum_block_kernel,
        grid=(grid,),
        in_specs=[pl.BlockSpec((_BLOCK_ROWS, _COLS), lambda i: (i, 0))],
        out_specs=pl.BlockSpec((1, _COLS), lambda i: (0, 0)),
        out_shape=jax.ShapeDtypeStruct((1, _COLS), jnp.float32),
    )(x)
    return out
